# Initial kernel scaffold; baseline (speedup 1.0000x reference)
import jax
import jax.numpy as jnp
from jax.experimental import pallas as pl


def kernel(x, w):
    raise NotImplementedError("write your pallas kernel here")


# speedup vs baseline: 1.5476x; 1.5476x over previous
import functools

import jax
import jax.numpy as jnp
from jax import lax
from jax.experimental import pallas as pl
from jax.experimental.pallas import tpu as pltpu
from jax.experimental.pallas import tpu_sc as plsc

_NC = 2
_NS = 16
_NW = _NC * _NS

_CH = 1024


def _build_emb(B, D, b_per_w):
    n_chunks = b_per_w // _CH
    mesh = plsc.VectorSubcoreMesh(core_axis_name="c", subcore_axis_name="s")

    @functools.partial(
        pl.kernel,
        out_type=jax.ShapeDtypeStruct((B, D), jnp.float32),
        mesh=mesh,
        scratch_types=[
            pltpu.VMEM((_CH,), jnp.int32),
            pltpu.VMEM((_CH, D), jnp.float32),
            pltpu.SemaphoreType.DMA,
        ],
        compiler_params=pltpu.CompilerParams(use_tc_tiling_on_sc=False),
    )
    def emb(idx_hbm, w_hbm, out_hbm, idx_v, rows_v, sem):
        wid = lax.axis_index("s") * _NC + lax.axis_index("c")
        base = wid * b_per_w

        def step(i, carry):
            off = base + i * _CH
            pltpu.sync_copy(idx_hbm.at[pl.ds(off, _CH)], idx_v)
            pltpu.async_copy(w_hbm.at[idx_v], rows_v, sem).wait()
            pltpu.sync_copy(rows_v, out_hbm.at[pl.ds(off, _CH)])
            return carry

        lax.fori_loop(0, n_chunks, step, 0)

    return emb


def kernel(x, w):
    B0, B1 = x.shape
    V, D = w.shape
    B = B0 * B1
    idx = x.reshape(B).astype(jnp.int32)
    assert B % (_NW * _CH) == 0
    b_per_w = B // _NW
    out = _build_emb(B, D, b_per_w)(idx, w)
    return out.reshape(B0, B1, D)


# speedup vs baseline: 1.5769x; 1.0189x over previous
import functools

import jax
import jax.numpy as jnp
from jax import lax
from jax.experimental import pallas as pl
from jax.experimental.pallas import tpu as pltpu
from jax.experimental.pallas import tpu_sc as plsc

_NC = 2
_NS = 16
_NW = _NC * _NS

_CH = 832
_NB = 4


def _build_emb(B, D, b_per_w):
    n_chunks = b_per_w // _CH
    mesh = plsc.VectorSubcoreMesh(core_axis_name="c", subcore_axis_name="s")

    @functools.partial(
        pl.kernel,
        out_type=jax.ShapeDtypeStruct((B, D), jnp.float32),
        mesh=mesh,
        scratch_types=[
            pltpu.VMEM((n_chunks, _CH), jnp.int32),
            pltpu.VMEM((_NB, _CH, D), jnp.float32),
            pltpu.SemaphoreType.DMA((_NB,)),
            pltpu.SemaphoreType.DMA((_NB,)),
        ],
        compiler_params=pltpu.CompilerParams(use_tc_tiling_on_sc=False),
    )
    def emb(idx_hbm, w_hbm, out_hbm, idx_v, rows_v, gsem, wsem):
        wid = lax.axis_index("s") * _NC + lax.axis_index("c")
        base = wid * b_per_w

        pltpu.sync_copy(idx_hbm.at[pl.ds(wid * n_chunks, n_chunks)], idx_v)

        def gather(i):
            b = i % _NB
            return pltpu.async_copy(
                w_hbm.at[idx_v.at[i]], rows_v.at[b], gsem.at[b])

        gathers = [gather(i) for i in range(_NB)]
        writes = [None] * n_chunks
        for i in range(n_chunks):
            b = i % _NB
            gathers[b].wait()
            writes[i] = pltpu.async_copy(
                rows_v.at[b], out_hbm.at[pl.ds(base + i * _CH, _CH)],
                wsem.at[b])
            if i + _NB < n_chunks:
                writes[i].wait()
                gathers[b] = gather(i + _NB)
        for i in range(n_chunks - _NB, n_chunks):
            writes[i].wait()

    return emb


def kernel(x, w):
    B0, B1 = x.shape
    V, D = w.shape
    B = B0 * B1
    assert B % (_NW * _CH) == 0
    b_per_w = B // _NW
    idx = x.reshape(B // _CH, _CH).astype(jnp.int32)
    out = _build_emb(B, D, b_per_w)(idx, w)
    return out.reshape(B0, B1, D)
